# baseline (speedup 1.0000x reference)
import functools

import jax
import jax.numpy as jnp
from jax import lax
from jax.experimental import pallas as pl
from jax.experimental.pallas import tpu as pltpu
from jax.experimental.pallas import tpu_sc as plsc

N_USERS = 1000000
DIM = 64
BATCH = 16384
L = 16
NC = 2
NS = 16
NW = NC * NS
WIN = 128
NFULL = N_USERS // WIN
TAIL_BASE = NFULL * WIN
TAIL_N = N_USERS - TAIL_BASE
MAPB = 1000448
MAPDUMP = 1000192
DUMP = BATCH * DIM
OUT1D = DUMP + 2048
HITROWS = 320
PATROWS = 16
CCAP = 1024
WPER = NFULL // NW
WEXTRA = NFULL - WPER * NW

_mesh = plsc.VectorSubcoreMesh(core_axis_name="c", subcore_axis_name="s")


@functools.partial(
    pl.kernel,
    mesh=_mesh,
    out_type=jax.ShapeDtypeStruct((MAPB,), jnp.int32),
    scratch_types=[
        pltpu.VMEM((BATCH,), jnp.int32),
        pltpu.VMEM((CCAP + L,), jnp.int32),
        pltpu.VMEM((CCAP + L,), jnp.int32),
        pltpu.VMEM((8, WIN), jnp.int32),
        pltpu.VMEM((8, WIN), jnp.int32),
        pltpu.VMEM((2048,), jnp.int32),
        pltpu.SemaphoreType.DMA,
    ],
    compiler_params=pltpu.CompilerParams(needs_layout_passes=False),
)
def _map_kernel(idx_hbm, map_hbm, idxfull_v, cidx_v, cpos_v,
                cidx2_v, cpos2_v, neg_v, sem_c):
    cid = lax.axis_index("c")
    sid = lax.axis_index("s")
    wid = sid * NC + cid
    lane = lax.iota(jnp.int32, L)

    pltpu.sync_copy(idx_hbm, idxfull_v)

    for i in range(2048 // L):
        neg_v[pl.ds(i * L, L)] = jnp.full((L,), -1, jnp.int32)
    rgn = MAPB // NW
    for k in range(15):
        pltpu.async_copy(
            neg_v, map_hbm.at[pl.ds(pl.multiple_of(wid * rgn + k * 2048, 8),
                                    2048)], sem_c)
    pltpu.async_copy(
        neg_v.at[pl.ds(0, 544)],
        map_hbm.at[pl.ds(pl.multiple_of(wid * rgn + 15 * 2048, 8), 544)],
        sem_c)
    for k in range(15):
        pltpu.make_async_copy(neg_v, map_hbm.at[pl.ds(0, 2048)], sem_c).wait()
    pltpu.make_async_copy(
        neg_v.at[pl.ds(0, 544)], map_hbm.at[pl.ds(0, 544)], sem_c).wait()

    for i in range((CCAP + L) // L):
        cidx_v[pl.ds(i * L, L)] = jnp.full((L,), MAPDUMP, jnp.int32)
        cpos_v[pl.ds(i * L, L)] = jnp.full((L,), -1, jnp.int32)

    lo = wid * rgn
    hi = jnp.where(wid == NW - 1, N_USERS, (wid + 1) * rgn)

    def cbody(k, nacc):
        iv = idxfull_v[pl.ds(k * L, L)]
        own = (iv >= lo) & (iv < hi)
        jv = k * L + lane
        plsc.store_compressed(cidx_v.at[pl.ds(nacc, L)], iv, mask=own)
        plsc.store_compressed(cpos_v.at[pl.ds(nacc, L)], jv, mask=own)
        cnt = jnp.max(plsc.all_reduce_population_count(own))
        return nacc + cnt

    lax.fori_loop(0, 1, cbody, 0)

    for r in range(8):
        for c in range(WIN // L):
            cidx2_v[r, pl.ds(c * L, L)] = cidx_v[pl.ds(r * WIN + c * L, L)]
            cpos2_v[r, pl.ds(c * L, L)] = cpos_v[pl.ds(r * WIN + c * L, L)]
    for r in range(8):
        pltpu.sync_copy(cpos2_v.at[r], map_hbm.at[cidx2_v.at[r]])


@functools.partial(
    pl.kernel,
    mesh=_mesh,
    out_type=jax.ShapeDtypeStruct((OUT1D,), jnp.float32),
    scratch_types=[
        pltpu.VMEM((BATCH,), jnp.int32),
        pltpu.VMEM((2, DIM, WIN), jnp.float32),
        pltpu.VMEM((DIM, TAIL_N), jnp.float32),
        pltpu.VMEM((2, WIN), jnp.int32),
        pltpu.VMEM((WIN,), jnp.int32),
        pltpu.VMEM((HITROWS, WIN), jnp.float32),
        pltpu.VMEM((HITROWS, WIN), jnp.int32),
        pltpu.VMEM((PATROWS, WIN), jnp.int32),
        pltpu.VMEM((PATROWS, WIN), jnp.int32),
        pltpu.VMEM((PATROWS, WIN), jnp.float32),
        pltpu.VMEM((WIN,), jnp.int32),
        pltpu.SemaphoreType.DMA,
        pltpu.SemaphoreType.DMA,
        pltpu.SemaphoreType.DMA,
    ],
    compiler_params=pltpu.CompilerParams(needs_layout_passes=False),
)
def _sweep_kernel(idx_hbm, wt_hbm, map_hbm, out_hbm, idxfull_v, win_v,
                  win64_v, mseg_v, mseg64_v, hitv_v, hitp_v,
                  pgat_v, psca_v, pval_v, mapchk_v, sem_b0, sem_b1, sem_s):
    cid = lax.axis_index("c")
    sid = lax.axis_index("s")
    wid = sid * NC + cid
    lane = lax.iota(jnp.int32, L)

    pltpu.sync_copy(idx_hbm, idxfull_v)

    wlo = wid * WPER + jnp.minimum(wid, WEXTRA)
    nwn = (WPER + jnp.where(wid < WEXTRA, 1, 0)) * 0 + 64

    def fetch(w, b, semb):
        woff = pl.multiple_of(w * WIN, WIN)
        pltpu.async_copy(wt_hbm.at[:, pl.ds(woff, WIN)], win_v.at[b], semb)
        pltpu.async_copy(map_hbm.at[pl.ds(woff, WIN)], mseg_v.at[b], semb)

    def wait_fetch(b, semb):
        pltpu.make_async_copy(
            wt_hbm.at[:, pl.ds(0, WIN)], win_v.at[b], semb).wait()
        pltpu.make_async_copy(
            map_hbm.at[pl.ds(0, WIN)], mseg_v.at[b], semb).wait()

    def extract_subchunk(win_ref, mv, s, nh):
        mask = mv >= 0

        def cond(st):
            m, _ = st
            return jnp.any(m)

        def body(st):
            m, n = st
            p = plsc.all_reduce_ffs(m)
            j = jnp.max(jnp.where(lane == p, mv, -1))
            jsp = jnp.full((L,), j, jnp.int32)
            lsp = p + s * L
            row = n // 2
            col = (n % 2) * DIM

            for c2 in range(DIM // L):
                dv = c2 * L + lane
                v = plsc.load_gather(win_ref, [dv, lsp])
                hitv_v[row, pl.ds(col + c2 * L, L)] = jnp.maximum(v, 0.01 * v)
                hitp_v[row, pl.ds(col + c2 * L, L)] = jsp * DIM + dv

            @pl.when(n % 2 == 0)
            def _():
                for c2 in range(DIM // L):
                    hitp_v[row, pl.ds(DIM + c2 * L, L)] = DUMP + c2 * L + lane

            return m & (lane != p), n + 1

        _, nh = lax.while_loop(cond, body, (mask, nh))
        return nh

    def process(b, nh):
        return nh + 1

    fetch(wlo, 0, sem_b0)

    def pair_body(p2, nh):
        w0 = wlo + 2 * p2

        @pl.when(2 * p2 + 1 < nwn)
        def _():
            fetch(w0 + 1, 1, sem_b1)

        wait_fetch(0, sem_b0)
        nh = process(0, nh)

        @pl.when(2 * p2 + 2 < nwn)
        def _():
            fetch(w0 + 2, 0, sem_b0)

        wait_fetch(1, sem_b1)
        nh = process(1, nh)
        return nh

    nhit = lax.fori_loop(0, nwn // 2, pair_body, 0)

    def odd_tail(nh):
        wait_fetch(0, sem_b0)
        return process(0, nh)

    nhit = lax.cond(nwn % 2 == 1, odd_tail, lambda nh: nh, nhit)

    pltpu.sync_copy(wt_hbm.at[:, pl.ds(TAIL_BASE, TAIL_N)], win64_v)
    pltpu.sync_copy(map_hbm.at[pl.ds(TAIL_BASE, WIN)], mseg64_v)
    for s in range(TAIL_N // L):
        mv = mseg64_v[pl.ds(s * L, L)]
        nhit = extract_subchunk(win64_v, mv, s, nhit)

    nrows = (nhit + 1) // 2 * 0

    def sc_body(r, carry):
        pltpu.async_copy(hitv_v.at[r], out_hbm.at[hitp_v.at[r]], sem_s)

        @pl.when(r >= 16)
        def _():
            pltpu.make_async_copy(
                hitv_v.at[0], out_hbm.at[hitp_v.at[0]], sem_s).wait()

        return carry

    lax.fori_loop(0, nrows, sc_body, 0)

    def drain_body(r, carry):
        pltpu.make_async_copy(
            hitv_v.at[0], out_hbm.at[hitp_v.at[0]], sem_s).wait()
        return carry

    lax.fori_loop(0, jnp.minimum(nrows, 16), drain_body, 0)

    whi = wlo + nwn
    own_tail = wid == NW - 1

    def patch_chunk(k, npat):
        pltpu.sync_copy(
            map_hbm.at[idxfull_v.at[pl.ds(k * WIN, WIN)]], mapchk_v)
        for s in range(WIN // L):
            mv = mapchk_v[pl.ds(s * L, L)]
            iv = idxfull_v[pl.ds(k * WIN + s * L, L)]
            jv = k * WIN + s * L + lane
            wv = lax.shift_right_logical(iv, 7)
            inr = (wv >= wlo) & (wv < whi)
            inr = inr | ((wv == NFULL) & own_tail)
            loser = inr & (mv != jv) & (jv < 0)

            def cond(st):
                m, _ = st
                return jnp.any(m)

            def body(st):
                m, n = st
                p = plsc.all_reduce_ffs(m)
                w = jnp.max(jnp.where(lane == p, mv, -1))
                wsp = jnp.full((L,), w, jnp.int32)
                j = k * WIN + s * L + jnp.max(jnp.where(lane == p, lane, -1))
                jsp = jnp.full((L,), j, jnp.int32)
                row = n // 2
                col = (n % 2) * DIM
                for c2 in range(DIM // L):
                    dv = c2 * L + lane
                    pgat_v[row, pl.ds(col + c2 * L, L)] = wsp * DIM + dv
                    psca_v[row, pl.ds(col + c2 * L, L)] = jsp * DIM + dv

                @pl.when(n % 2 == 0)
                def _():
                    for c2 in range(DIM // L):
                        dmp = DUMP + c2 * L + lane
                        pgat_v[row, pl.ds(DIM + c2 * L, L)] = dmp
                        psca_v[row, pl.ds(DIM + c2 * L, L)] = dmp

                return m & (lane != p), n + 1

            _, npat = lax.while_loop(cond, body, (loser, npat))
        return npat

    npat = lax.fori_loop(0, 1, patch_chunk, 0)
    nprow = (npat + 1) // 2 * 0

    def pg_body(r, carry):
        pltpu.sync_copy(out_hbm.at[pgat_v.at[r]], pval_v.at[r])
        return carry

    lax.fori_loop(0, nprow, pg_body, 0)

    def ps_body(r, carry):
        pltpu.sync_copy(pval_v.at[r], out_hbm.at[psca_v.at[r]])
        return carry

    lax.fori_loop(0, nprow, ps_body, 0)


def kernel(user_idx, W):
    idx32 = user_idx.astype(jnp.int32)
    mp = _map_kernel(idx32)
    out1d = _sweep_kernel(idx32, W.T, mp)
    return out1d[:BATCH * DIM].reshape(BATCH, DIM)


# speedup vs baseline: 7.9085x; 7.9085x over previous
import functools

import jax
import jax.numpy as jnp
from jax import lax
from jax.experimental import pallas as pl
from jax.experimental.pallas import tpu as pltpu
from jax.experimental.pallas import tpu_sc as plsc

N_USERS = 1000000
DIM = 64
BATCH = 16384
LANES = 16
NUM_CORES = 2
NUM_SUBCORES = 16
NUM_WORKERS = NUM_CORES * NUM_SUBCORES
BPW = BATCH // NUM_WORKERS

_mesh = plsc.VectorSubcoreMesh(core_axis_name="c", subcore_axis_name="s")


@functools.partial(
    pl.kernel,
    mesh=_mesh,
    out_type=jax.ShapeDtypeStruct((BATCH, DIM), jnp.float32),
    scratch_types=[
        pltpu.VMEM((BPW,), jnp.int32),
        pltpu.VMEM((BPW, DIM), jnp.float32),
        pltpu.SemaphoreType.DMA,
    ],
    compiler_params=pltpu.CompilerParams(use_tc_tiling_on_sc=False),
)
def _gather_lrelu(idx_hbm, table_hbm, out_hbm, idx_v, rows_v, sem):
    wid = lax.axis_index("s") * NUM_CORES + lax.axis_index("c")
    base = wid * BPW
    pltpu.sync_copy(idx_hbm.at[pl.ds(base, BPW)], idx_v)
    pltpu.async_copy(table_hbm.at[idx_v], rows_v, sem).wait()

    def body(i, carry):
        for c in range(DIM // LANES):
            v = rows_v[i, pl.ds(c * LANES, LANES)]
            rows_v[i, pl.ds(c * LANES, LANES)] = jnp.maximum(v, 0.01 * v)
        return carry

    lax.fori_loop(0, BPW, body, 0)
    pltpu.sync_copy(rows_v, out_hbm.at[pl.ds(base, BPW)])


def kernel(user_idx, W):
    return _gather_lrelu(user_idx.astype(jnp.int32), W)
